# baseline (speedup 1.0000x reference)
import jax
import jax.numpy as jnp
from jax import lax
from jax.experimental import pallas as pl
from jax.experimental.pallas import tpu as pltpu
from jax.experimental.pallas import tpu_sc as plsc

N_NODES = 10000
N_EDGES = 320000
D_FEAT = 128

NUM_CORES = 2
NUM_SUBCORES = 16
NUM_WORKERS = NUM_CORES * NUM_SUBCORES
EDGES_PER_WORKER = N_EDGES // NUM_WORKERS
CHUNK = 80
NUM_CHUNKS = EDGES_PER_WORKER // CHUNK
BLOCKS_PER_CHUNK = CHUNK // 16
K_UNROLL = 16
NACC = 4
NBUF = 4
MAIN_CHUNKS = (NUM_CHUNKS // NBUF - 1) * NBUF


def _dot_chunk(urows, vrows, outbuf, out_off):
    lanes = lax.iota(jnp.int32, 16)
    npairs = D_FEAT // 2
    for b in range(BLOCKS_PER_CHUNK):
        rows = lanes + (16 * b)

        def k_body(i, accs):
            lo_acc, hi_acc = accs
            for u in range(K_UNROLL):
                col = (lanes + (i * K_UNROLL + u)) & (npairs - 1)
                uv = plsc.bitcast(plsc.load_gather(urows, [rows, col]),
                                  jnp.bfloat16)
                vv = plsc.bitcast(plsc.load_gather(vrows, [rows, col]),
                                  jnp.bfloat16)
                lo, hi = plsc.unpack(uv * vv,
                                     format=plsc.PackFormat.INTERLEAVED)
                lo_acc = lo_acc + lo
                hi_acc = hi_acc + hi
            return lo_acc, hi_acc

        zero = jnp.zeros((16,), jnp.float32)
        lo_acc, hi_acc = lax.fori_loop(0, npairs // K_UNROLL, k_body,
                                       (zero, zero))
        outbuf[pl.ds(out_off + 16 * b, 16)] = lo_acc + hi_acc


def _sc_kernel(h_hbm, src_hbm, dst_hbm, out_hbm,
               srcbuf, dstbuf, ubufs, vbufs, outbuf, usems, vsems):
    wid = lax.axis_index("s") * NUM_CORES + lax.axis_index("c")
    wbase = wid * EDGES_PER_WORKER
    pltpu.sync_copy(src_hbm.at[pl.ds(wbase, EDGES_PER_WORKER)], srcbuf)
    pltpu.sync_copy(dst_hbm.at[pl.ds(wbase, EDGES_PER_WORKER)], dstbuf)

    def gather_pair(c, s):
        off = c * CHUNK
        pltpu.async_copy(
            h_hbm.at[srcbuf.at[pl.ds(off, CHUNK)]], ubufs[s], usems[s])
        pltpu.async_copy(
            h_hbm.at[dstbuf.at[pl.ds(off, CHUNK)]], vbufs[s], vsems[s])

    def wait_pair(c, s):
        off = c * CHUNK
        pltpu.make_async_copy(
            h_hbm.at[srcbuf.at[pl.ds(off, CHUNK)]], ubufs[s], usems[s]).wait()
        pltpu.make_async_copy(
            h_hbm.at[dstbuf.at[pl.ds(off, CHUNK)]], vbufs[s], vsems[s]).wait()

    for s in range(NBUF - 1):
        gather_pair(s, s)

    def body(g, carry):
        c_base = NBUF * g
        for s in range(NBUF):
            c = c_base + s
            wait_pair(c, s)
            gather_pair(c + NBUF - 1, (s + NBUF - 1) % NBUF)
            _dot_chunk(ubufs[s], vbufs[s], outbuf, c * CHUNK)
        return carry

    lax.fori_loop(0, MAIN_CHUNKS // NBUF, body, 0)
    for c in range(MAIN_CHUNKS, NUM_CHUNKS):
        s = c % NBUF
        wait_pair(c, s)
        if c + NBUF - 1 < NUM_CHUNKS:
            gather_pair(c + NBUF - 1, (s + NBUF - 1) % NBUF)
        _dot_chunk(ubufs[s], vbufs[s], outbuf, c * CHUNK)

    pltpu.sync_copy(outbuf, out_hbm.at[pl.ds(wbase, EDGES_PER_WORKER)])


@jax.jit
def kernel(h, edge_index):
    h_packed = jax.lax.bitcast_convert_type(
        h.astype(jnp.bfloat16).reshape(N_NODES, D_FEAT // 2, 2),
        jnp.int32)
    src = edge_index[0]
    dst = edge_index[1]
    mesh = plsc.VectorSubcoreMesh(core_axis_name="c", subcore_axis_name="s")
    k = pl.kernel(
        _sc_kernel,
        out_type=jax.ShapeDtypeStruct((N_EDGES,), jnp.float32),
        mesh=mesh,
        compiler_params=pltpu.CompilerParams(
            use_tc_tiling_on_sc=False, needs_layout_passes=False),
        scratch_types=[
            pltpu.VMEM((EDGES_PER_WORKER,), jnp.int32),
            pltpu.VMEM((EDGES_PER_WORKER,), jnp.int32),
            [pltpu.VMEM((CHUNK, D_FEAT // 2), jnp.int32)
             for _ in range(NBUF)],
            [pltpu.VMEM((CHUNK, D_FEAT // 2), jnp.int32)
             for _ in range(NBUF)],
            pltpu.VMEM((EDGES_PER_WORKER,), jnp.float32),
            [pltpu.SemaphoreType.DMA for _ in range(NBUF)],
            [pltpu.SemaphoreType.DMA for _ in range(NBUF)],
        ],
    )
    return k(h_packed, src, dst)


# speedup vs baseline: 1.4840x; 1.4840x over previous
import jax
import jax.numpy as jnp
from jax import lax
from jax.experimental import pallas as pl
from jax.experimental.pallas import tpu as pltpu
from jax.experimental.pallas import tpu_sc as plsc

N_NODES = 10000
N_EDGES = 320000
D_FEAT = 128

NUM_CORES = 2
NUM_SUBCORES = 16
NUM_WORKERS = NUM_CORES * NUM_SUBCORES
EDGES_PER_WORKER = N_EDGES // NUM_WORKERS
CHUNK = 80
NUM_CHUNKS = EDGES_PER_WORKER // CHUNK
BLOCKS_PER_CHUNK = CHUNK // 16
K_UNROLL = 4
NACC = 4
NBUF = 4
MAIN_CHUNKS = (NUM_CHUNKS // NBUF - 1) * NBUF


def _dot_chunk(urows, vrows, outbuf, out_off):
    lanes = lax.iota(jnp.int32, 16)
    npairs = D_FEAT // 2
    for b in range(BLOCKS_PER_CHUNK):
        rows = lanes + (16 * b)

        def k_body(i, accs):
            lo_acc, hi_acc = accs
            for u in range(K_UNROLL):
                col = (lanes + (i * K_UNROLL + u)) & (npairs - 1)
                uv = plsc.bitcast(plsc.load_gather(urows, [rows, col]),
                                  jnp.bfloat16)
                vv = plsc.bitcast(plsc.load_gather(vrows, [rows, col]),
                                  jnp.bfloat16)
                lo, hi = plsc.unpack(uv * vv,
                                     format=plsc.PackFormat.INTERLEAVED)
                lo_acc = lo_acc + lo
                hi_acc = hi_acc + hi
            return lo_acc, hi_acc

        zero = jnp.zeros((16,), jnp.float32)
        lo_acc, hi_acc = lax.fori_loop(0, npairs // K_UNROLL, k_body,
                                       (zero, zero))
        outbuf[pl.ds(out_off + 16 * b, 16)] = lo_acc + hi_acc


def _sc_kernel(h_hbm, src_hbm, dst_hbm, out_hbm,
               srcbuf, dstbuf, ubufs, vbufs, outbuf, usems, vsems):
    wid = lax.axis_index("s") * NUM_CORES + lax.axis_index("c")
    wbase = wid * EDGES_PER_WORKER
    pltpu.sync_copy(src_hbm.at[pl.ds(wbase, EDGES_PER_WORKER)], srcbuf)
    pltpu.sync_copy(dst_hbm.at[pl.ds(wbase, EDGES_PER_WORKER)], dstbuf)

    def gather_pair(c, s):
        off = c * CHUNK
        pltpu.async_copy(
            h_hbm.at[srcbuf.at[pl.ds(off, CHUNK)]], ubufs[s], usems[s])
        pltpu.async_copy(
            h_hbm.at[dstbuf.at[pl.ds(off, CHUNK)]], vbufs[s], vsems[s])

    def wait_pair(c, s):
        off = c * CHUNK
        pltpu.make_async_copy(
            h_hbm.at[srcbuf.at[pl.ds(off, CHUNK)]], ubufs[s], usems[s]).wait()
        pltpu.make_async_copy(
            h_hbm.at[dstbuf.at[pl.ds(off, CHUNK)]], vbufs[s], vsems[s]).wait()

    for s in range(NBUF - 1):
        gather_pair(s, s)

    def body(g, carry):
        c_base = NBUF * g
        for s in range(NBUF):
            c = c_base + s
            wait_pair(c, s)
            gather_pair(c + NBUF - 1, (s + NBUF - 1) % NBUF)
            _dot_chunk(ubufs[s], vbufs[s], outbuf, c * CHUNK)
        return carry

    lax.fori_loop(0, MAIN_CHUNKS // NBUF, body, 0)
    for c in range(MAIN_CHUNKS, NUM_CHUNKS):
        s = c % NBUF
        wait_pair(c, s)
        if c + NBUF - 1 < NUM_CHUNKS:
            gather_pair(c + NBUF - 1, (s + NBUF - 1) % NBUF)
        _dot_chunk(ubufs[s], vbufs[s], outbuf, c * CHUNK)

    pltpu.sync_copy(outbuf, out_hbm.at[pl.ds(wbase, EDGES_PER_WORKER)])


@jax.jit
def kernel(h, edge_index):
    h_packed = jax.lax.bitcast_convert_type(
        h.astype(jnp.bfloat16).reshape(N_NODES, D_FEAT // 2, 2),
        jnp.int32)
    src = edge_index[0]
    dst = edge_index[1]
    mesh = plsc.VectorSubcoreMesh(core_axis_name="c", subcore_axis_name="s")
    k = pl.kernel(
        _sc_kernel,
        out_type=jax.ShapeDtypeStruct((N_EDGES,), jnp.float32),
        mesh=mesh,
        compiler_params=pltpu.CompilerParams(
            use_tc_tiling_on_sc=False, needs_layout_passes=False),
        scratch_types=[
            pltpu.VMEM((EDGES_PER_WORKER,), jnp.int32),
            pltpu.VMEM((EDGES_PER_WORKER,), jnp.int32),
            [pltpu.VMEM((CHUNK, D_FEAT // 2), jnp.int32)
             for _ in range(NBUF)],
            [pltpu.VMEM((CHUNK, D_FEAT // 2), jnp.int32)
             for _ in range(NBUF)],
            pltpu.VMEM((EDGES_PER_WORKER,), jnp.float32),
            [pltpu.SemaphoreType.DMA for _ in range(NBUF)],
            [pltpu.SemaphoreType.DMA for _ in range(NBUF)],
        ],
    )
    return k(h_packed, src, dst)
